# baseline (speedup 1.0000x reference)
import functools

import jax
import jax.numpy as jnp
from jax import lax
from jax.experimental import pallas as pl
from jax.experimental.pallas import tpu as pltpu
from jax.experimental.pallas import tpu_sc as plsc

N_EMB = 8192
DIM = 256
N_TOK = 8192
TOK_BLK = 512
N_BLK = N_TOK // TOK_BLK


def _vq_body(x_ref, emb_ref, enc_ref, idx_ref, loss_ref, ppl_ref,
             e2_ref, cnt_ref, acc_ref):
    i = pl.program_id(0)
    emb = emb_ref[...]

    @pl.when(i == 0)
    def _init():
        ones = jnp.ones((1, DIM), jnp.float32)
        e2_ref[...] = lax.dot_general(
            ones, emb * emb, (((1,), (1,)), ((), ())),
            preferred_element_type=jnp.float32)

    xb = x_ref[...]
    nmm = lax.dot_general(-2.0 * xb, emb, (((1,), (1,)), ((), ())),
                          preferred_element_type=jnp.float32)
    x2 = jnp.sum(xb * xb, axis=1, keepdims=True)
    dist = (x2 + e2_ref[...]) + nmm
    half = N_EMB // 2
    d0 = dist[:, :half]
    d1 = dist[:, half:]
    m0 = jnp.min(d0, axis=1, keepdims=True)
    m1 = jnp.min(d1, axis=1, keepdims=True)
    iota_h = lax.broadcasted_iota(jnp.int32, (1, half), 1)
    big = jnp.int32(N_EMB)
    i0 = jnp.min(jnp.where(d0 == m0, iota_h, big), axis=1, keepdims=True)
    i1 = jnp.min(jnp.where(d1 == m1, iota_h, big), axis=1, keepdims=True) + half
    h0 = m0.astype(jnp.bfloat16).astype(jnp.float32) <= m1
    idx = jnp.where(h0, i0, i1)
    dmin = jnp.where(h0, m0, m1)
    iota = lax.broadcasted_iota(jnp.int32, (1, N_EMB), 1)
    idx_ref[...] = idx
    onehot = jnp.where(iota == idx, 1.0, 0.0).astype(jnp.float32)
    enc_ref[...] = onehot

    col = jnp.sum(onehot, axis=0, keepdims=True)
    part = jnp.sum(dmin, keepdims=True)

    @pl.when(i == 0)
    def _acc0():
        cnt_ref[...] = col
        acc_ref[...] = part

    @pl.when(i > 0)
    def _acc():
        cnt_ref[...] += col
        acc_ref[...] += part

    @pl.when(i == N_BLK - 1)
    def _fin():
        loss_ref[...] = acc_ref[...] * (1.25 / float(N_TOK * DIM))
        avg = cnt_ref[...] * (1.0 / float(N_TOK))
        ent = jnp.sum(avg * jnp.log(avg + 1e-10), keepdims=True)
        ppl_ref[...] = jnp.exp(-ent)


def _vq_tc(flat, embedding):
    return pl.pallas_call(
        _vq_body,
        grid=(N_BLK,),
        in_specs=[
            pl.BlockSpec((TOK_BLK, DIM), lambda i: (i, 0)),
            pl.BlockSpec((N_EMB, DIM), lambda i: (0, 0)),
        ],
        out_specs=[
            pl.BlockSpec((TOK_BLK, N_EMB), lambda i: (i, 0)),
            pl.BlockSpec((TOK_BLK, 1), lambda i: (i, 0)),
            pl.BlockSpec((1, 1), lambda i: (0, 0)),
            pl.BlockSpec((1, 1), lambda i: (0, 0)),
        ],
        out_shape=[
            jax.ShapeDtypeStruct((N_TOK, N_EMB), jnp.float32),
            jax.ShapeDtypeStruct((N_TOK, 1), jnp.int32),
            jax.ShapeDtypeStruct((1, 1), jnp.float32),
            jax.ShapeDtypeStruct((1, 1), jnp.float32),
        ],
        scratch_shapes=[
            pltpu.VMEM((1, N_EMB), jnp.float32),
            pltpu.VMEM((1, N_EMB), jnp.float32),
            pltpu.VMEM((1, 1), jnp.float32),
        ],
        compiler_params=pltpu.CompilerParams(
            dimension_semantics=("arbitrary",),
            vmem_limit_bytes=128 * 1024 * 1024,
        ),
    )(flat, embedding)


def _sc_gather(embedding, idx_flat):
    info = plsc.get_sparse_core_info()
    nc, ns = info.num_cores, info.num_subcores
    nw = nc * ns
    bpw = N_TOK // nw
    mesh = plsc.VectorSubcoreMesh(core_axis_name="c", subcore_axis_name="s")

    @functools.partial(
        pl.kernel,
        mesh=mesh,
        out_type=jax.ShapeDtypeStruct((N_TOK, DIM), jnp.float32),
        scratch_types=[
            pltpu.VMEM((bpw,), jnp.int32),
            pltpu.VMEM((bpw, DIM), jnp.float32),
            pltpu.SemaphoreType.DMA,
        ],
    )
    def _gather(emb_hbm, idx_hbm, out_hbm, idx_v, rows_v, sem):
        wid = lax.axis_index("s") * nc + lax.axis_index("c")
        base = wid * bpw
        pltpu.sync_copy(idx_hbm.at[pl.ds(base, bpw)], idx_v)
        pltpu.async_copy(emb_hbm.at[idx_v], rows_v, sem).wait()
        pltpu.sync_copy(rows_v, out_hbm.at[pl.ds(base, bpw)])

    return _gather(embedding, idx_flat)


def kernel(inputs, embedding):
    x = jnp.transpose(inputs, (0, 2, 3, 1))
    flat = x.reshape(-1, DIM)
    enc, idx2d, loss11, ppl11 = _vq_tc(flat, embedding)
    q_flat = _sc_gather(embedding, idx2d.reshape(N_TOK))
    q = q_flat.reshape(x.shape)
    quantized_st = x + (q - x)
    return (loss11[0, 0], jnp.transpose(quantized_st, (0, 3, 1, 2)),
            ppl11[0, 0], enc)


# speedup vs baseline: 1.0763x; 1.0763x over previous
import functools

import jax
import jax.numpy as jnp
from jax import lax
from jax.experimental import pallas as pl
from jax.experimental.pallas import tpu as pltpu
from jax.experimental.pallas import tpu_sc as plsc

N_EMB = 8192
DIM = 256
N_TOK = 8192
TOK_BLK = 256
N_BLK = N_TOK // TOK_BLK


def _vq_body(x_ref, emb_ref, enc_ref, idx_ref, loss_ref, ppl_ref,
             e2_ref, cnt_ref, acc_ref):
    i = pl.program_id(0)
    emb = emb_ref[...]

    @pl.when(i == 0)
    def _init():
        ones = jnp.ones((1, DIM), jnp.float32)
        e2_ref[...] = lax.dot_general(
            ones, emb * emb, (((1,), (1,)), ((), ())),
            preferred_element_type=jnp.float32)

    xb = x_ref[...]
    nmm = lax.dot_general(-2.0 * xb, emb, (((1,), (1,)), ((), ())),
                          preferred_element_type=jnp.float32)
    x2 = jnp.sum(xb * xb, axis=1, keepdims=True)
    dist = (x2 + e2_ref[...]) + nmm
    half = N_EMB // 2
    d0 = dist[:, :half]
    d1 = dist[:, half:]
    m0 = jnp.min(d0, axis=1, keepdims=True)
    m1 = jnp.min(d1, axis=1, keepdims=True)
    i0 = jnp.argmin(d0, axis=1).astype(jnp.int32)[:, None]
    i1 = jnp.argmin(d1, axis=1).astype(jnp.int32)[:, None] + half
    h0 = m0.astype(jnp.bfloat16).astype(jnp.float32) <= m1
    idx = jnp.where(h0, i0, i1)
    dmin = jnp.where(h0, m0, m1)
    iota = lax.broadcasted_iota(jnp.int32, (1, N_EMB), 1)
    idx_ref[...] = idx
    onehot = jnp.where(iota == idx, 1.0, 0.0).astype(jnp.float32)
    enc_ref[...] = onehot

    ones_row = jnp.ones((1, TOK_BLK), jnp.float32)
    col = lax.dot_general(ones_row, onehot, (((1,), (0,)), ((), ())),
                          preferred_element_type=jnp.float32)
    part = lax.dot_general(ones_row, dmin, (((1,), (0,)), ((), ())),
                           preferred_element_type=jnp.float32)

    @pl.when(i == 0)
    def _acc0():
        cnt_ref[...] = col
        acc_ref[...] = part

    @pl.when(i > 0)
    def _acc():
        cnt_ref[...] += col
        acc_ref[...] += part

    @pl.when(i == N_BLK - 1)
    def _fin():
        loss_ref[...] = acc_ref[...] * (1.25 / float(N_TOK * DIM))
        avg = cnt_ref[...] * (1.0 / float(N_TOK))
        ent = jnp.sum(avg * jnp.log(avg + 1e-10), keepdims=True)
        ppl_ref[...] = jnp.exp(-ent)


def _vq_tc(flat, embedding):
    return pl.pallas_call(
        _vq_body,
        grid=(N_BLK,),
        in_specs=[
            pl.BlockSpec((TOK_BLK, DIM), lambda i: (i, 0)),
            pl.BlockSpec((N_EMB, DIM), lambda i: (0, 0)),
        ],
        out_specs=[
            pl.BlockSpec((TOK_BLK, N_EMB), lambda i: (i, 0)),
            pl.BlockSpec((TOK_BLK, 1), lambda i: (i, 0)),
            pl.BlockSpec((1, 1), lambda i: (0, 0)),
            pl.BlockSpec((1, 1), lambda i: (0, 0)),
        ],
        out_shape=[
            jax.ShapeDtypeStruct((N_TOK, N_EMB), jnp.float32),
            jax.ShapeDtypeStruct((N_TOK, 1), jnp.int32),
            jax.ShapeDtypeStruct((1, 1), jnp.float32),
            jax.ShapeDtypeStruct((1, 1), jnp.float32),
        ],
        scratch_shapes=[
            pltpu.VMEM((1, N_EMB), jnp.float32),
            pltpu.VMEM((1, N_EMB), jnp.float32),
            pltpu.VMEM((1, 1), jnp.float32),
        ],
        compiler_params=pltpu.CompilerParams(
            dimension_semantics=("arbitrary",),
            vmem_limit_bytes=128 * 1024 * 1024,
        ),
    )(flat, embedding)


def _sc_gather(embedding, idx_flat):
    info = plsc.get_sparse_core_info()
    nc, ns = info.num_cores, info.num_subcores
    nw = nc * ns
    bpw = N_TOK // nw
    mesh = plsc.VectorSubcoreMesh(core_axis_name="c", subcore_axis_name="s")

    @functools.partial(
        pl.kernel,
        mesh=mesh,
        out_type=jax.ShapeDtypeStruct((N_TOK, DIM), jnp.float32),
        scratch_types=[
            pltpu.VMEM((bpw,), jnp.int32),
            pltpu.VMEM((bpw, DIM), jnp.float32),
            pltpu.SemaphoreType.DMA,
        ],
    )
    def _gather(emb_hbm, idx_hbm, out_hbm, idx_v, rows_v, sem):
        wid = lax.axis_index("s") * nc + lax.axis_index("c")
        base = wid * bpw
        pltpu.sync_copy(idx_hbm.at[pl.ds(base, bpw)], idx_v)
        pltpu.async_copy(emb_hbm.at[idx_v], rows_v, sem).wait()
        pltpu.sync_copy(rows_v, out_hbm.at[pl.ds(base, bpw)])

    return _gather(embedding, idx_flat)


def kernel(inputs, embedding):
    x = jnp.transpose(inputs, (0, 2, 3, 1))
    flat = x.reshape(-1, DIM)
    enc, idx2d, loss11, ppl11 = _vq_tc(flat, embedding)
    q_flat = _sc_gather(embedding, idx2d.reshape(N_TOK))
    q = q_flat.reshape(x.shape)
    quantized_st = x + (q - x)
    return (loss11[0, 0], jnp.transpose(quantized_st, (0, 3, 1, 2)),
            ppl11[0, 0], enc)
